# initial kernel scaffold (unmeasured)
import jax
import jax.numpy as jnp
from jax import lax
from jax.experimental import pallas as pl
from jax.experimental.pallas import tpu as pltpu

N_DEV = 16
BLK = 256


def kernel(x, w_mat):
    m_total, k_per = x.shape
    k_total, n = w_mat.shape
    assert m_total == N_DEV * BLK and k_per == BLK

    def body(x_ref, w_ref, out_ref, xg_ref, amax_ref,
             x_send_sems, x_recv_sems, a_send_sems, a_recv_sems):
        my = lax.axis_index("i")

        barrier_sem = pltpu.get_barrier_semaphore()
        for d in range(1, N_DEV):
            peer = lax.rem(my + d, N_DEV)
            pl.semaphore_signal(
                barrier_sem, inc=1,
                device_id=(peer,), device_id_type=pl.DeviceIdType.MESH,
            )
        pl.semaphore_wait(barrier_sem, N_DEV - 1)

        rdmas = []
        for d in range(1, N_DEV):
            peer = lax.rem(my + d, N_DEV)
            rdma = pltpu.make_async_remote_copy(
                src_ref=x_ref.at[pl.ds(peer * BLK, BLK), :],
                dst_ref=xg_ref.at[d],
                send_sem=x_send_sems.at[d],
                recv_sem=x_recv_sems.at[d],
                device_id=(peer,),
                device_id_type=pl.DeviceIdType.MESH,
            )
            rdma.start()
            rdmas.append(rdma)

        def wblock(j):
            return w_ref[pl.ds(j * BLK, BLK), :]

        y = jnp.dot(
            x_ref[pl.ds(my * BLK, BLK), :], wblock(my),
            preferred_element_type=jnp.float32,
        )
        for d in range(1, N_DEV):
            rdmas[d - 1].wait_recv()
            j = lax.rem(my - d + N_DEV, N_DEV)
            y = y + jnp.dot(
                xg_ref[d], wblock(j), preferred_element_type=jnp.float32
            )
        y = jnp.maximum(y, 0.0)

        local_amax = jnp.max(y)
        amax_ref[0] = jnp.full((8, 128), local_amax, jnp.float32)
        a_rdmas = []
        for d in range(1, N_DEV):
            peer = lax.rem(my + d, N_DEV)
            a = pltpu.make_async_remote_copy(
                src_ref=amax_ref.at[0],
                dst_ref=amax_ref.at[d],
                send_sem=a_send_sems.at[d],
                recv_sem=a_recv_sems.at[d],
                device_id=(peer,),
                device_id_type=pl.DeviceIdType.MESH,
            )
            a.start()
            a_rdmas.append(a)
        for a in a_rdmas:
            a.wait_recv()
        gmax = jnp.max(amax_ref[...])

        scale = gmax / 127.0
        q = jnp.clip(jnp.round(y / scale), -127.0, 127.0)
        out_ref[...] = q * scale

        for r in rdmas:
            r.wait_send()
        for a in a_rdmas:
            a.wait_send()

    return pl.pallas_call(
        body,
        out_shape=jax.ShapeDtypeStruct((BLK, n), jnp.float32),
        in_specs=[
            pl.BlockSpec(memory_space=pltpu.VMEM),
            pl.BlockSpec(memory_space=pltpu.VMEM),
        ],
        out_specs=pl.BlockSpec(memory_space=pltpu.VMEM),
        scratch_shapes=[
            pltpu.VMEM((N_DEV, BLK, BLK), jnp.bfloat16),
            pltpu.VMEM((N_DEV, 8, 128), jnp.float32),
            pltpu.SemaphoreType.DMA((N_DEV,)),
            pltpu.SemaphoreType.DMA((N_DEV,)),
            pltpu.SemaphoreType.DMA((N_DEV,)),
            pltpu.SemaphoreType.DMA((N_DEV,)),
        ],
        compiler_params=pltpu.CompilerParams(collective_id=0),
    )(x, w_mat)


# baseline (device time: 106211 ns/iter reference)
import jax
import jax.numpy as jnp
from jax import lax
from jax.experimental import pallas as pl
from jax.experimental.pallas import tpu as pltpu

N_DEV = 16
BLK = 256


def _a2a_gather(x):
    m_total, k_per = x.shape

    def body(x_ref, out_ref, xbf_ref, send_sems, recv_sems):
        my = lax.axis_index("i")
        xbf_ref[...] = x_ref[...].astype(jnp.bfloat16)
        rdmas = []
        for d in range(1, N_DEV):
            peer = lax.rem(my + d, N_DEV)
            rdma = pltpu.make_async_remote_copy(
                src_ref=xbf_ref.at[pl.ds(peer * BLK, BLK), :],
                dst_ref=out_ref.at[d],
                send_sem=send_sems.at[d],
                recv_sem=recv_sems.at[d],
                device_id=(peer,),
                device_id_type=pl.DeviceIdType.MESH,
            )
            rdma.start()
            rdmas.append(rdma)
        for r in rdmas:
            r.wait_recv()
        for r in rdmas:
            r.wait_send()

    return pl.pallas_call(
        body,
        out_shape=jax.ShapeDtypeStruct((N_DEV, BLK, BLK), jnp.bfloat16),
        in_specs=[pl.BlockSpec(memory_space=pltpu.VMEM)],
        out_specs=pl.BlockSpec(memory_space=pltpu.VMEM),
        scratch_shapes=[
            pltpu.VMEM((m_total, BLK), jnp.bfloat16),
            pltpu.SemaphoreType.DMA((N_DEV,)),
            pltpu.SemaphoreType.DMA((N_DEV,)),
        ],
    )(x)


def _gemm(xs, w_mat):
    k_total, n = w_mat.shape

    def body(xs_ref, w_ref, y_ref, amax_ref):
        j = pl.program_id(0)
        wbf = w_ref[...].astype(jnp.bfloat16)
        part = jnp.dot(xs_ref[0], wbf, preferred_element_type=jnp.float32)

        @pl.when(j == 0)
        def _():
            y_ref[...] = part

        @pl.when(j > 0)
        def _():
            y_ref[...] = y_ref[...] + part

        @pl.when(j == N_DEV - 1)
        def _():
            yr = jnp.maximum(y_ref[...], 0.0)
            y_ref[...] = yr
            amax_ref[...] = jnp.full((8, 128), jnp.max(yr), jnp.float32)

    return pl.pallas_call(
        body,
        grid=(N_DEV,),
        out_shape=(
            jax.ShapeDtypeStruct((BLK, n), jnp.float32),
            jax.ShapeDtypeStruct((8, 128), jnp.float32),
        ),
        in_specs=[
            pl.BlockSpec((1, BLK, BLK), lambda j: (j, 0, 0)),
            pl.BlockSpec((BLK, n), lambda j: (j, 0)),
        ],
        out_specs=(
            pl.BlockSpec((BLK, n), lambda j: (0, 0)),
            pl.BlockSpec((8, 128), lambda j: (0, 0)),
        ),
        compiler_params=pltpu.CompilerParams(
            dimension_semantics=("arbitrary",),
        ),
    )(xs, w_mat)


def _quant(y, amax_tile):
    m, n = y.shape

    def body(y_ref, a_ref, out_ref, gat_ref, send_sems, recv_sems):
        my = lax.axis_index("i")
        rdmas = []
        for d in range(1, N_DEV):
            peer = lax.rem(my + d, N_DEV)
            rdma = pltpu.make_async_remote_copy(
                src_ref=a_ref,
                dst_ref=gat_ref.at[d],
                send_sem=send_sems.at[d],
                recv_sem=recv_sems.at[d],
                device_id=(peer,),
                device_id_type=pl.DeviceIdType.MESH,
            )
            rdma.start()
            rdmas.append(rdma)
        gmax = jnp.max(a_ref[...])
        for d in range(1, N_DEV):
            rdmas[d - 1].wait_recv()
            gmax = jnp.maximum(gmax, jnp.max(gat_ref[d]))
        scale = gmax / 127.0
        q = jnp.clip(jnp.round(y_ref[...] / scale), -127.0, 127.0)
        out_ref[...] = q * scale
        for r in rdmas:
            r.wait_send()

    return pl.pallas_call(
        body,
        out_shape=jax.ShapeDtypeStruct((m, n), jnp.float32),
        in_specs=[
            pl.BlockSpec(memory_space=pltpu.VMEM),
            pl.BlockSpec(memory_space=pltpu.VMEM),
        ],
        out_specs=pl.BlockSpec(memory_space=pltpu.VMEM),
        scratch_shapes=[
            pltpu.VMEM((N_DEV, 8, 128), jnp.float32),
            pltpu.SemaphoreType.DMA((N_DEV,)),
            pltpu.SemaphoreType.DMA((N_DEV,)),
        ],
    )(y, amax_tile)


def kernel(x, w_mat):
    m_total, k_per = x.shape
    assert m_total == N_DEV * BLK and k_per == BLK

    my = lax.axis_index("i")

    slots = _a2a_gather(x)

    idx = jnp.mod(my - jnp.arange(N_DEV), N_DEV)
    by_sender = jnp.take(slots, idx, axis=0)
    own = lax.dynamic_slice(x, (my * BLK, 0), (BLK, BLK))
    own = own.astype(jnp.bfloat16)[None]
    by_sender = lax.dynamic_update_slice(by_sender, own, (my, 0, 0))

    y, amax_tile = _gemm(by_sender, w_mat)
    return _quant(y, amax_tile)
